# baseline (speedup 1.0000x reference)
import dataclasses
import functools

import jax
import jax.numpy as jnp
from jax.experimental import pallas as pl
from jax.experimental.pallas import tpu as pltpu
from jax.experimental.pallas import tpu_sc as plsc

_T, _D, _H, _E, _K = 64, 1024, 2048, 64, 2
_HC = 2048
_L = 16
_NSUB = 16
_TPW = _T // _NSUB
_NCH = _E // _L
_NEG = -3.0e38


def _logits_kernel(x_ref, Wr_ref, br_ref, Wn_ref, bn_ref, noise_ref, noisy_ref):
    x = x_ref[...]
    logits = jnp.dot(x, Wr_ref[...], preferred_element_type=jnp.float32) + br_ref[...]
    nl = jnp.dot(x, Wn_ref[...], preferred_element_type=jnp.float32) + bn_ref[...]
    noisy_ref[...] = logits + noise_ref[...] * jax.nn.softplus(nl)


def _top2(chunks, idxs):
    m = chunks[0]
    for c in range(1, _NCH):
        m = jnp.maximum(m, chunks[c])
    m0 = jnp.max(m)
    cm = jnp.where(chunks[0] == m0, idxs[0], _E)
    for c in range(1, _NCH):
        cm = jnp.minimum(cm, jnp.where(chunks[c] == m0, idxs[c], _E))
    i0 = jnp.min(cm)
    masked = [jnp.where(idxs[c] == i0, _NEG, chunks[c]) for c in range(_NCH)]
    m2 = masked[0]
    for c in range(1, _NCH):
        m2 = jnp.maximum(m2, masked[c])
    m1 = jnp.max(m2)
    cm1 = jnp.where(masked[0] == m1, idxs[0], _E)
    for c in range(1, _NCH):
        cm1 = jnp.minimum(cm1, jnp.where(masked[c] == m1, idxs[c], _E))
    i1 = jnp.min(cm1)
    return m0, i0, m1, i1


def _sc_router_kernel(noisy_hbm, G_hbm, ids_hbm, n_hbm,
                      row_v, g_v, big_v, ids_v, n_v):
    core = jax.lax.axis_index("c")
    sub = jax.lax.axis_index("s")
    idxs = [jax.lax.iota(jnp.int32, _L) + c * _L for c in range(_NCH)]

    @pl.when(core == 1)
    def _work():
        base = sub * _TPW
        pltpu.sync_copy(noisy_hbm.at[pl.ds(base, _TPW)], row_v)
        for t in range(_TPW):
            chunks = [row_v[t, pl.ds(c * _L, _L)] for c in range(_NCH)]
            m0, i0, m1, i1 = _top2(chunks, idxs)
            rv = jnp.exp(jnp.broadcast_to(m1 - m0, (_L,)))
            g0 = 1.0 / (1.0 + rv)
            g1 = rv / (1.0 + rv)
            for c in range(_NCH):
                g_v[t, pl.ds(c * _L, _L)] = (
                    jnp.where(idxs[c] == i0, g0, 0.0)
                    + jnp.where(idxs[c] == i1, g1, 0.0))
        pltpu.sync_copy(g_v, G_hbm.at[pl.ds(base, _TPW)])

    @pl.when((core == 0) & (sub == 0))
    def _compact():
        pltpu.sync_copy(noisy_hbm, big_v)
        accs = [jnp.zeros((_L,), jnp.int32) for _ in range(_NCH)]
        for t in range(_T):
            chunks = [big_v[t, pl.ds(c * _L, _L)] for c in range(_NCH)]
            _, i0, _, i1 = _top2(chunks, idxs)
            for c in range(_NCH):
                accs[c] = (accs[c]
                           + (idxs[c] == i0).astype(jnp.int32)
                           + (idxs[c] == i1).astype(jnp.int32))
        carry = jnp.int32(0)
        last = jnp.int32(0)
        poss, masks = [], []
        for c in range(_NCH):
            amc = (accs[c] > 0).astype(jnp.int32)
            cumc = plsc.cumsum(amc) + carry
            carry = jnp.max(cumc)
            poss.append(cumc - 1)
            masks.append(amc > 0)
            last = jnp.maximum(last, jnp.max(jnp.where(amc > 0, idxs[c], -1)))
        lastv = jnp.broadcast_to(last, (_L,))
        for c in range(_NCH):
            ids_v[pl.ds(c * _L, _L)] = lastv
        for c in range(_NCH):
            plsc.store_scatter(ids_v, [poss[c]], idxs[c], mask=masks[c])
        n_v[...] = jnp.broadcast_to(carry, (_L,))
        pltpu.sync_copy(ids_v, ids_hbm)
        pltpu.sync_copy(n_v, n_hbm)


def _make_sc_router():
    return functools.partial(
        pl.kernel,
        out_type=[
            jax.ShapeDtypeStruct((_T, _E), jnp.float32),
            jax.ShapeDtypeStruct((_E,), jnp.int32),
            jax.ShapeDtypeStruct((_L,), jnp.int32),
        ],
        mesh=plsc.VectorSubcoreMesh(core_axis_name="c", subcore_axis_name="s"),
        compiler_params=dataclasses.replace(
            pltpu.CompilerParams(), needs_layout_passes=False),
        scratch_types=[
            pltpu.VMEM((_TPW, _E), jnp.float32),
            pltpu.VMEM((_TPW, _E), jnp.float32),
            pltpu.VMEM((_T, _E), jnp.float32),
            pltpu.VMEM((_E,), jnp.int32),
            pltpu.VMEM((_L,), jnp.int32),
        ],
    )(_sc_router_kernel)


def _ffn_kernel(ids_ref, n_ref, x_ref, G_ref, w1_ref, w3_ref, w2_ref, out_ref):
    j = pl.program_id(0)

    @pl.when(j == 0)
    def _init():
        out_ref[...] = jnp.zeros_like(out_ref)

    @pl.when(j < n_ref[0])
    def _body():
        xb = x_ref[...].astype(jnp.bfloat16)
        hp = jnp.dot(xb, w1_ref[0].astype(jnp.bfloat16),
                     preferred_element_type=jnp.float32)
        gp = jnp.dot(xb, w3_ref[0].astype(jnp.bfloat16),
                     preferred_element_type=jnp.float32)
        s = (hp * jax.nn.sigmoid(hp) * gp).astype(jnp.bfloat16)
        y = jnp.dot(s, w2_ref[0].astype(jnp.bfloat16),
                    preferred_element_type=jnp.float32)
        e = ids_ref[j]
        ecols = jax.lax.broadcasted_iota(jnp.int32, (_T, _E), 1)
        gcol = jnp.sum(jnp.where(ecols == e, G_ref[...], 0.0),
                       axis=1, keepdims=True)
        out_ref[...] += y * gcol


def kernel(x, Wr, br, Wn, bn, w1, w2, w3):
    noise = jax.random.normal(jax.random.key(1234), (_T, _E), dtype=jnp.float32)
    noisy = pl.pallas_call(
        _logits_kernel,
        out_shape=jax.ShapeDtypeStruct((_T, _E), jnp.float32),
    )(x, Wr, br.reshape(1, _E), Wn, bn.reshape(1, _E), noise)

    G, ids, n16 = _make_sc_router()(noisy)
    n = n16[0:1]

    grid = (_E,)
    out = pl.pallas_call(
        _ffn_kernel,
        grid_spec=pltpu.PrefetchScalarGridSpec(
            num_scalar_prefetch=2,
            grid=grid,
            in_specs=[
                pl.BlockSpec((_T, _D), lambda j, ids, n: (0, 0)),
                pl.BlockSpec((_T, _E), lambda j, ids, n: (0, 0)),
                pl.BlockSpec((1, _D, _HC), lambda j, ids, n: (ids[j], 0, 0)),
                pl.BlockSpec((1, _D, _HC), lambda j, ids, n: (ids[j], 0, 0)),
                pl.BlockSpec((1, _HC, _D), lambda j, ids, n: (ids[j], 0, 0)),
            ],
            out_specs=pl.BlockSpec((_T, _D), lambda j, ids, n: (0, 0)),
        ),
        out_shape=jax.ShapeDtypeStruct((_T, _D), jnp.float32),
        compiler_params=pltpu.CompilerParams(
            dimension_semantics=("arbitrary",),
        ),
    )(ids, n, x, G, w1, w3, w2)
    return out


# speedup vs baseline: 1.0218x; 1.0218x over previous
import dataclasses
import functools

import jax
import jax.numpy as jnp
from jax.experimental import pallas as pl
from jax.experimental.pallas import tpu as pltpu
from jax.experimental.pallas import tpu_sc as plsc

_T, _D, _H, _E, _K = 64, 1024, 2048, 64, 2
_HC = 2048
_L = 16
_NCH = _E // _L


def _router_tc_kernel(x_ref, Wr_ref, br_ref, Wn_ref, bn_ref, noise_ref,
                      G_ref, am_ref):
    x = x_ref[...]
    logits = jnp.dot(x, Wr_ref[...], preferred_element_type=jnp.float32) + br_ref[...]
    nl = jnp.dot(x, Wn_ref[...], preferred_element_type=jnp.float32) + bn_ref[...]
    noisy = logits + noise_ref[...] * jax.nn.softplus(nl)

    ecols = jax.lax.broadcasted_iota(jnp.int32, (_T, _E), 1)
    m0 = jnp.max(noisy, axis=1, keepdims=True)
    i0 = jnp.min(jnp.where(noisy == m0, ecols, _E), axis=1, keepdims=True)
    masked = jnp.where(ecols == i0, -jnp.inf, noisy)
    m1 = jnp.max(masked, axis=1, keepdims=True)
    i1 = jnp.min(jnp.where(masked == m1, ecols, _E), axis=1, keepdims=True)
    r = jnp.exp(m1 - m0)
    g0 = 1.0 / (1.0 + r)
    g1 = r / (1.0 + r)
    G_ref[...] = jnp.where(ecols == i0, g0, 0.0) + jnp.where(ecols == i1, g1, 0.0)
    sel = ((ecols == i0) | (ecols == i1)).astype(jnp.int32)
    am_ref[...] = jnp.max(sel, axis=0, keepdims=True)


def _sc_compact_kernel(am_hbm, ids_hbm, n_hbm, am_v, ids_v, n_v):
    core = jax.lax.axis_index("c")
    sub = jax.lax.axis_index("s")

    @pl.when((core == 0) & (sub == 0))
    def _compact():
        pltpu.sync_copy(am_hbm, am_v)
        idxs = [jax.lax.iota(jnp.int32, _L) + c * _L for c in range(_NCH)]
        carry = jnp.int32(0)
        last = jnp.int32(0)
        poss, masks = [], []
        for c in range(_NCH):
            amc = am_v[pl.ds(c * _L, _L)]
            cumc = plsc.cumsum(amc) + carry
            carry = jnp.max(cumc)
            poss.append(cumc - 1)
            masks.append(amc > 0)
            last = jnp.maximum(last, jnp.max(jnp.where(amc > 0, idxs[c], -1)))
        lastv = jnp.broadcast_to(last, (_L,))
        for c in range(_NCH):
            ids_v[pl.ds(c * _L, _L)] = lastv
        for c in range(_NCH):
            plsc.store_scatter(ids_v, [poss[c]], idxs[c], mask=masks[c])
        n_v[...] = jnp.broadcast_to(carry, (_L,))
        pltpu.sync_copy(ids_v, ids_hbm)
        pltpu.sync_copy(n_v, n_hbm)


def _make_sc_compact():
    return functools.partial(
        pl.kernel,
        out_type=[
            jax.ShapeDtypeStruct((_E,), jnp.int32),
            jax.ShapeDtypeStruct((_L,), jnp.int32),
        ],
        mesh=plsc.VectorSubcoreMesh(core_axis_name="c", subcore_axis_name="s"),
        compiler_params=dataclasses.replace(
            pltpu.CompilerParams(), needs_layout_passes=False),
        scratch_types=[
            pltpu.VMEM((_E,), jnp.int32),
            pltpu.VMEM((_E,), jnp.int32),
            pltpu.VMEM((_L,), jnp.int32),
        ],
    )(_sc_compact_kernel)


def _ffn_kernel(ids_ref, n_ref, x_ref, G_ref, w1_ref, w3_ref, w2_ref, out_ref):
    j = pl.program_id(0)

    @pl.when(j == 0)
    def _init():
        out_ref[...] = jnp.zeros_like(out_ref)

    @pl.when(j < n_ref[0])
    def _body():
        xb = x_ref[...].astype(jnp.bfloat16)
        hp = jnp.dot(xb, w1_ref[0].astype(jnp.bfloat16),
                     preferred_element_type=jnp.float32)
        gp = jnp.dot(xb, w3_ref[0].astype(jnp.bfloat16),
                     preferred_element_type=jnp.float32)
        s = (hp * jax.nn.sigmoid(hp) * gp).astype(jnp.bfloat16)
        y = jnp.dot(s, w2_ref[0].astype(jnp.bfloat16),
                    preferred_element_type=jnp.float32)
        e = ids_ref[j]
        ecols = jax.lax.broadcasted_iota(jnp.int32, (_T, _E), 1)
        gcol = jnp.sum(jnp.where(ecols == e, G_ref[...], 0.0),
                       axis=1, keepdims=True)
        out_ref[...] += y * gcol


def kernel(x, Wr, br, Wn, bn, w1, w2, w3):
    noise = jax.random.normal(jax.random.key(1234), (_T, _E), dtype=jnp.float32)
    G, am2d = pl.pallas_call(
        _router_tc_kernel,
        out_shape=[
            jax.ShapeDtypeStruct((_T, _E), jnp.float32),
            jax.ShapeDtypeStruct((1, _E), jnp.int32),
        ],
    )(x, Wr, br.reshape(1, _E), Wn, bn.reshape(1, _E), noise)

    ids, n16 = _make_sc_compact()(am2d.reshape(_E))
    n = n16[0:1]

    out = pl.pallas_call(
        _ffn_kernel,
        grid_spec=pltpu.PrefetchScalarGridSpec(
            num_scalar_prefetch=2,
            grid=(_E,),
            in_specs=[
                pl.BlockSpec((_T, _D), lambda j, ids, n: (0, 0)),
                pl.BlockSpec((_T, _E), lambda j, ids, n: (0, 0)),
                pl.BlockSpec((1, _D, _HC), lambda j, ids, n: (ids[j], 0, 0)),
                pl.BlockSpec((1, _D, _HC), lambda j, ids, n: (ids[j], 0, 0)),
                pl.BlockSpec((1, _HC, _D), lambda j, ids, n: (ids[j], 0, 0)),
            ],
            out_specs=pl.BlockSpec((_T, _D), lambda j, ids, n: (0, 0)),
        ),
        out_shape=jax.ShapeDtypeStruct((_T, _D), jnp.float32),
        compiler_params=pltpu.CompilerParams(
            dimension_semantics=("arbitrary",),
        ),
    )(ids, n, x, G, w1, w3, w2)
    return out
